# baseline (speedup 1.0000x reference)
import functools

import jax
import jax.numpy as jnp
from jax import lax
from jax.experimental import pallas as pl
from jax.experimental.pallas import tpu as pltpu
from jax.experimental.pallas import tpu_sc as plsc

_INTERP = False
_SC_PARAMS = pltpu.CompilerParams(needs_layout_passes=False)

NWORDS = 512
CS = 10240
BLK = 256
NBLK = CS // BLK
NSEL = CS // 10
NTILES = 32
NQ = 4
NG = NTILES // NQ
CPG = CS // NG
QROWS = NWORDS // NQ
QELEMS = QROWS * NWORDS


def _sort_body(g_ref, start_ref, end_ref, perm_ref, pidx_ref, gold_ref, se_ref):
    keys = g_ref[...]
    lane = lax.broadcasted_iota(jnp.int32, (BLK, NWORDS), 1)
    pay = lane
    for k in range(1, 10):
        kbit = 1 << k
        up = (lane & kbit) == 0
        for j in range(k - 1, -1, -1):
            d = 1 << j
            is_upper = (lane & d) != 0
            pk = jnp.where(is_upper, jnp.roll(keys, d, axis=1),
                           jnp.roll(keys, -d, axis=1))
            pp = jnp.where(is_upper, jnp.roll(pay, d, axis=1),
                           jnp.roll(pay, -d, axis=1))
            own_gt = (keys > pk) | ((keys == pk) & (pay > pp))
            take = jnp.logical_xor(own_gt, jnp.logical_xor(up, is_upper))
            take = jnp.logical_not(take)
            keys = jnp.where(take, pk, keys)
            pay = jnp.where(take, pp, pay)

    perm_ref[...] = pay
    nxt = jnp.roll(pay, -1, axis=1)
    pidx_ref[...] = pay * NWORDS + nxt
    gold_ref[...] = jnp.all(pay == lane, axis=1, keepdims=True).astype(jnp.int32)
    start_b = start_ref[...]
    end_b = end_ref[...]
    se0 = jnp.sum(jnp.where(lane == pay[:, 0:1], start_b, 0.0), axis=1,
                  keepdims=True)
    se1 = jnp.sum(jnp.where(lane == pay[:, NWORDS - 1:NWORDS], end_b, 0.0),
                  axis=1, keepdims=True)
    se_ref[...] = se0 + se1


def _sort_stage(gumbel, start2, end2):
    return pl.pallas_call(
        _sort_body,
        grid=(NBLK,),
        in_specs=[
            pl.BlockSpec((BLK, NWORDS), lambda i: (i, 0)),
            pl.BlockSpec((1, NWORDS), lambda i: (0, 0)),
            pl.BlockSpec((1, NWORDS), lambda i: (0, 0)),
        ],
        out_specs=[
            pl.BlockSpec((BLK, NWORDS), lambda i: (i, 0)),
            pl.BlockSpec((BLK, NWORDS), lambda i: (i, 0)),
            pl.BlockSpec((BLK, 1), lambda i: (i, 0)),
            pl.BlockSpec((BLK, 1), lambda i: (i, 0)),
        ],
        out_shape=[
            jax.ShapeDtypeStruct((CS, NWORDS), jnp.int32),
            jax.ShapeDtypeStruct((CS, NWORDS), jnp.int32),
            jax.ShapeDtypeStruct((CS, 1), jnp.int32),
            jax.ShapeDtypeStruct((CS, 1), jnp.float32),
        ],
        interpret=_INTERP,
    )(gumbel, start2, end2)


SCCHUNK = 32


def _score_kernel(bigf_hbm, pidx_hbm, wpart_hbm, bq, ib0, ib1, wacc,
                  semt, sem0, sem1):
    wid = lax.axis_index("s") * 2 + lax.axis_index("c")
    q = wid % NQ
    g = wid // NQ
    base = q * QELEMS
    ct = pltpu.async_copy(bigf_hbm.at[pl.ds(base, QELEMS)], bq, semt)
    lanes = lax.iota(jnp.int32, 16)
    base_v = jnp.full((16,), base, jnp.int32)
    ct.wait()

    def process(ibuf, wacc_off):
        for cc in range(SCCHUNK // 16):
            lanes_c = lanes + 16 * cc

            def one_j(j, acc):
                jv = jnp.full((16,), 0, jnp.int32) + j
                iv = plsc.load_gather(ibuf, [lanes_c, jv])
                li = iv - base_v
                ok = (li >= 0) & (li < QELEMS)
                lic = li & (QELEMS - 1)
                v = plsc.load_gather(bq, [lic], mask=ok)
                return acc + jnp.where(ok, v, 0.0)

            def jblock(jb, acc):
                for jj in range(16):
                    acc = one_j(jb * 16 + jj, acc)
                return acc

            acc = jnp.zeros((16,), jnp.float32)
            acc = lax.fori_loop(0, 31, jblock, acc)
            for jj in range(496, 511):
                acc = one_j(jj, acc)
            wacc[pl.ds(wacc_off + cc * 16, 16)] = acc

    @pl.loop(0, CPG // (2 * SCCHUNK))
    def _pair(h):
        cbase = g * CPG + h * 2 * SCCHUNK
        cpa = pltpu.async_copy(pidx_hbm.at[pl.ds(cbase, SCCHUNK)], ib0, sem0)
        cpb = pltpu.async_copy(pidx_hbm.at[pl.ds(cbase + SCCHUNK, SCCHUNK)],
                               ib1, sem1)
        cpa.wait()
        process(ib0, h * 2 * SCCHUNK)
        cpb.wait()
        process(ib1, h * 2 * SCCHUNK + SCCHUNK)

    pltpu.sync_copy(wacc, wpart_hbm.at[q, pl.ds(g * CPG, CPG)])


def _score_stage(bigf, pidx):
    mesh = plsc.VectorSubcoreMesh(core_axis_name="c", subcore_axis_name="s",
                                  num_cores=2, num_subcores=16)
    k = pl.kernel(
        _score_kernel,
        out_type=jax.ShapeDtypeStruct((NQ, CS), jnp.float32),
        mesh=mesh,
        scratch_types=[
            pltpu.VMEM((QELEMS,), jnp.float32),
            pltpu.VMEM((SCCHUNK, NWORDS), jnp.int32),
            pltpu.VMEM((SCCHUNK, NWORDS), jnp.int32),
            pltpu.VMEM((CPG,), jnp.float32),
            pltpu.SemaphoreType.DMA,
            pltpu.SemaphoreType.DMA,
            pltpu.SemaphoreType.DMA,
        ],
        compiler_params=_SC_PARAMS,
        interpret=_INTERP,
    )
    return k(bigf, pidx)


def _prep_body(wpart_ref, se_ref, gold_ref, u_ref, t_ref, w_ref, idx0_ref):
    wp = wpart_ref[...]
    w = wp[0:1] + wp[1:2] + wp[2:3] + wp[3:4] + se_ref[...]
    lu = jnp.log(u_ref[...])
    lane = lax.broadcasted_iota(jnp.int32, (1, CS), 1)
    gold = gold_ref[...] != 0
    neg_inf = jnp.float32(-jnp.inf)
    t = jnp.where(gold | (lane == 0), neg_inf, w - lu)
    t_ref[...] = t
    w_ref[...] = w
    idx0 = jnp.min(jnp.where(gold, CS, lane))
    idx0_ref[0, 0] = jnp.where(idx0 == CS, 0, idx0).astype(jnp.int32)


def _prep_stage(wpart, se_row, gold_row, u_row):
    return pl.pallas_call(
        _prep_body,
        out_shape=[
            jax.ShapeDtypeStruct((1, CS), jnp.float32),
            jax.ShapeDtypeStruct((1, CS), jnp.float32),
            jax.ShapeDtypeStruct((1, 1), jnp.int32),
        ],
        in_specs=[
            pl.BlockSpec((NQ, CS), lambda: (0, 0)),
            pl.BlockSpec((1, CS), lambda: (0, 0)),
            pl.BlockSpec((1, CS), lambda: (0, 0)),
            pl.BlockSpec((1, CS), lambda: (0, 0)),
        ],
        out_specs=[
            pl.BlockSpec((1, CS), lambda: (0, 0)),
            pl.BlockSpec((1, CS), lambda: (0, 0)),
            pl.BlockSpec(memory_space=pltpu.MemorySpace.SMEM),
        ],
        interpret=_INTERP,
    )(wpart, se_row, gold_row, u_row)


def _scan_body(t_ref, w_ref, idx0_ref, sel_ref):
    a0 = idx0_ref[0, 0]
    w0 = w_ref[0, 0]

    def outer(s, carry):
        a, wl = carry
        for r in range(10):
            i = s * 10 + r
            ti = t_ref[0, i]
            acc = ti > wl
            a = jnp.where(acc, i, a)
            wl = jnp.where(acc, w_ref[0, i], wl)
            if r == 9:
                sel_ref[0, s] = a
        return a, wl

    lax.fori_loop(0, NSEL, outer, (a0, w0))


def _scan_stage(t_row, w_row, idx0):
    return pl.pallas_call(
        _scan_body,
        out_shape=jax.ShapeDtypeStruct((1, NSEL), jnp.int32),
        in_specs=[
            pl.BlockSpec(memory_space=pltpu.MemorySpace.SMEM),
            pl.BlockSpec(memory_space=pltpu.MemorySpace.SMEM),
            pl.BlockSpec(memory_space=pltpu.MemorySpace.SMEM),
        ],
        out_specs=pl.BlockSpec(memory_space=pltpu.MemorySpace.SMEM),
        interpret=_INTERP,
    )(t_row, w_row, idx0)


def _rowgather_kernel(perm_hbm, sel_hbm, out_hbm, idx_v, rows_v, sem):
    wid = lax.axis_index("s") * 2 + lax.axis_index("c")
    b = wid * (NSEL // NTILES)
    pltpu.sync_copy(sel_hbm.at[pl.ds(b, NSEL // NTILES)], idx_v)
    pltpu.async_copy(perm_hbm.at[idx_v], rows_v, sem).wait()
    pltpu.sync_copy(rows_v, out_hbm.at[pl.ds(b, NSEL // NTILES)])


def _rowgather_stage(perm, sel):
    mesh = plsc.VectorSubcoreMesh(core_axis_name="c", subcore_axis_name="s",
                                  num_cores=2, num_subcores=16)
    k = pl.kernel(
        _rowgather_kernel,
        out_type=jax.ShapeDtypeStruct((NSEL, NWORDS), jnp.int32),
        mesh=mesh,
        scratch_types=[
            pltpu.VMEM((NSEL // NTILES,), jnp.int32),
            pltpu.VMEM((NSEL // NTILES, NWORDS), jnp.int32),
            pltpu.SemaphoreType.DMA,
        ],
        compiler_params=_SC_PARAMS,
        interpret=_INTERP,
    )
    return k(perm, sel)


def kernel(n_words, bigram, start, end, gumbel_noise, uniforms):
    start2 = start.reshape(1, NWORDS)
    end2 = end.reshape(1, NWORDS)
    perm, pidx, gold, se = _sort_stage(gumbel_noise, start2, end2)
    wpart = _score_stage(bigram.reshape(-1), pidx)
    t_row, w_row, idx0 = _prep_stage(wpart, se.reshape(1, CS),
                                     gold.reshape(1, CS),
                                     uniforms.reshape(1, CS))
    sel = _scan_stage(t_row, w_row, idx0)
    return _rowgather_stage(perm, sel.reshape(NSEL))


# speedup vs baseline: 1.1764x; 1.1764x over previous
import functools

import jax
import jax.numpy as jnp
from jax import lax
from jax.experimental import pallas as pl
from jax.experimental.pallas import tpu as pltpu
from jax.experimental.pallas import tpu_sc as plsc

_INTERP = False
_SC_PARAMS = pltpu.CompilerParams(needs_layout_passes=False)

NWORDS = 512
CS = 10240
BLK = 128
NBLK = CS // BLK
NSEL = CS // 10
NTILES = 32
NQ = 4
NG = NTILES // NQ
CPG = CS // NG
QROWS = NWORDS // NQ
QELEMS = QROWS * NWORDS


def _sort_body(g_ref, start_ref, end_ref, perm_ref, pidx_ref, gold_ref, se_ref):
    keys = g_ref[...]
    lane = lax.broadcasted_iota(jnp.int32, (BLK, NWORDS), 1)
    pay = lane
    for k in range(1, 10):
        kbit = 1 << k
        up = (lane & kbit) == 0
        for j in range(k - 1, -1, -1):
            d = 1 << j
            is_upper = (lane & d) != 0
            pk = jnp.where(is_upper, jnp.roll(keys, d, axis=1),
                           jnp.roll(keys, -d, axis=1))
            pp = jnp.where(is_upper, jnp.roll(pay, d, axis=1),
                           jnp.roll(pay, -d, axis=1))
            own_gt = (keys > pk) | ((keys == pk) & (pay > pp))
            take = jnp.logical_xor(own_gt, jnp.logical_xor(up, is_upper))
            take = jnp.logical_not(take)
            keys = jnp.where(take, pk, keys)
            pay = jnp.where(take, pp, pay)

    perm_ref[...] = pay
    nxt = jnp.roll(pay, -1, axis=1)
    pidx_ref[...] = pay * NWORDS + nxt
    gold_ref[...] = jnp.all(pay == lane, axis=1, keepdims=True).astype(jnp.int32)
    start_b = start_ref[...]
    end_b = end_ref[...]
    se0 = jnp.sum(jnp.where(lane == pay[:, 0:1], start_b, 0.0), axis=1,
                  keepdims=True)
    se1 = jnp.sum(jnp.where(lane == pay[:, NWORDS - 1:NWORDS], end_b, 0.0),
                  axis=1, keepdims=True)
    se_ref[...] = se0 + se1


def _sort_stage(gumbel, start2, end2):
    return pl.pallas_call(
        _sort_body,
        grid=(NBLK,),
        in_specs=[
            pl.BlockSpec((BLK, NWORDS), lambda i: (i, 0)),
            pl.BlockSpec((1, NWORDS), lambda i: (0, 0)),
            pl.BlockSpec((1, NWORDS), lambda i: (0, 0)),
        ],
        out_specs=[
            pl.BlockSpec((BLK, NWORDS), lambda i: (i, 0)),
            pl.BlockSpec((BLK, NWORDS), lambda i: (i, 0)),
            pl.BlockSpec((BLK, 1), lambda i: (i, 0)),
            pl.BlockSpec((BLK, 1), lambda i: (i, 0)),
        ],
        out_shape=[
            jax.ShapeDtypeStruct((CS, NWORDS), jnp.int32),
            jax.ShapeDtypeStruct((CS, NWORDS), jnp.int32),
            jax.ShapeDtypeStruct((CS, 1), jnp.int32),
            jax.ShapeDtypeStruct((CS, 1), jnp.float32),
        ],
        interpret=_INTERP,
    )(gumbel, start2, end2)


SCCHUNK = 32


def _score_kernel(bigf_hbm, pidx_hbm, wpart_hbm, bq, ib0, ib1, wacc,
                  semt, sem0, sem1):
    wid = lax.axis_index("s") * 2 + lax.axis_index("c")
    q = wid % NQ
    g = wid // NQ
    base = q * QELEMS
    ct = pltpu.async_copy(bigf_hbm.at[pl.ds(base, QELEMS)], bq, semt)
    lanes = lax.iota(jnp.int32, 16)
    base_v = jnp.full((16,), base, jnp.int32)
    ct.wait()

    def process(ibuf, wacc_off):
        for cc in range(SCCHUNK // 16):
            lanes_c = lanes + 16 * cc

            def one_j(j, acc):
                jv = jnp.full((16,), 0, jnp.int32) + j
                iv = plsc.load_gather(ibuf, [lanes_c, jv])
                li = iv - base_v
                ok = (li >= 0) & (li < QELEMS)
                lic = li & (QELEMS - 1)
                v = plsc.load_gather(bq, [lic], mask=ok)
                return acc + jnp.where(ok, v, 0.0)

            def jblock(jb, acc):
                for jj in range(16):
                    acc = one_j(jb * 16 + jj, acc)
                return acc

            acc = jnp.zeros((16,), jnp.float32)
            acc = lax.fori_loop(0, 31, jblock, acc)
            for jj in range(496, 511):
                acc = one_j(jj, acc)
            wacc[pl.ds(wacc_off + cc * 16, 16)] = acc

    @pl.loop(0, CPG // (2 * SCCHUNK))
    def _pair(h):
        cbase = g * CPG + h * 2 * SCCHUNK
        cpa = pltpu.async_copy(pidx_hbm.at[pl.ds(cbase, SCCHUNK)],
                               ib0.at[:, pl.ds(0, NWORDS)], sem0)
        cpb = pltpu.async_copy(pidx_hbm.at[pl.ds(cbase + SCCHUNK, SCCHUNK)],
                               ib1.at[:, pl.ds(0, NWORDS)], sem1)
        cpa.wait()
        process(ib0, h * 2 * SCCHUNK)
        cpb.wait()
        process(ib1, h * 2 * SCCHUNK + SCCHUNK)

    pltpu.sync_copy(wacc, wpart_hbm.at[q, pl.ds(g * CPG, CPG)])


def _score_stage(bigf, pidx):
    mesh = plsc.VectorSubcoreMesh(core_axis_name="c", subcore_axis_name="s",
                                  num_cores=2, num_subcores=16)
    k = pl.kernel(
        _score_kernel,
        out_type=jax.ShapeDtypeStruct((NQ, CS), jnp.float32),
        mesh=mesh,
        scratch_types=[
            pltpu.VMEM((QELEMS,), jnp.float32),
            pltpu.VMEM((SCCHUNK, NWORDS + 1), jnp.int32),
            pltpu.VMEM((SCCHUNK, NWORDS + 1), jnp.int32),
            pltpu.VMEM((CPG,), jnp.float32),
            pltpu.SemaphoreType.DMA,
            pltpu.SemaphoreType.DMA,
            pltpu.SemaphoreType.DMA,
        ],
        compiler_params=_SC_PARAMS,
        interpret=_INTERP,
    )
    return k(bigf, pidx)


def _prep_body(wpart_ref, se_ref, gold_ref, u_ref, t_ref, w_ref, idx0_ref):
    wp = wpart_ref[...]
    w = wp[0:1] + wp[1:2] + wp[2:3] + wp[3:4] + se_ref[...]
    lu = jnp.log(u_ref[...])
    lane = lax.broadcasted_iota(jnp.int32, (1, CS), 1)
    gold = gold_ref[...] != 0
    neg_inf = jnp.float32(-jnp.inf)
    t = jnp.where(gold | (lane == 0), neg_inf, w - lu)
    t_ref[...] = t
    w_ref[...] = w
    idx0 = jnp.min(jnp.where(gold, CS, lane))
    idx0_ref[0, 0] = jnp.where(idx0 == CS, 0, idx0).astype(jnp.int32)


def _prep_stage(wpart, se_row, gold_row, u_row):
    return pl.pallas_call(
        _prep_body,
        out_shape=[
            jax.ShapeDtypeStruct((1, CS), jnp.float32),
            jax.ShapeDtypeStruct((1, CS), jnp.float32),
            jax.ShapeDtypeStruct((1, 1), jnp.int32),
        ],
        in_specs=[
            pl.BlockSpec((NQ, CS), lambda: (0, 0)),
            pl.BlockSpec((1, CS), lambda: (0, 0)),
            pl.BlockSpec((1, CS), lambda: (0, 0)),
            pl.BlockSpec((1, CS), lambda: (0, 0)),
        ],
        out_specs=[
            pl.BlockSpec((1, CS), lambda: (0, 0)),
            pl.BlockSpec((1, CS), lambda: (0, 0)),
            pl.BlockSpec(memory_space=pltpu.MemorySpace.SMEM),
        ],
        interpret=_INTERP,
    )(wpart, se_row, gold_row, u_row)


def _scan_body(t_ref, w_ref, idx0_ref, sel_ref):
    a0 = idx0_ref[0, 0]
    w0 = w_ref[0, 0]

    def outer(s, carry):
        a, wl = carry
        for r in range(10):
            i = s * 10 + r
            ti = t_ref[0, i]
            acc = ti > wl
            a = jnp.where(acc, i, a)
            wl = jnp.where(acc, w_ref[0, i], wl)
            if r == 9:
                sel_ref[0, s] = a
        return a, wl

    lax.fori_loop(0, NSEL, outer, (a0, w0))


def _scan_stage(t_row, w_row, idx0):
    return pl.pallas_call(
        _scan_body,
        out_shape=jax.ShapeDtypeStruct((1, NSEL), jnp.int32),
        in_specs=[
            pl.BlockSpec(memory_space=pltpu.MemorySpace.SMEM),
            pl.BlockSpec(memory_space=pltpu.MemorySpace.SMEM),
            pl.BlockSpec(memory_space=pltpu.MemorySpace.SMEM),
        ],
        out_specs=pl.BlockSpec(memory_space=pltpu.MemorySpace.SMEM),
        interpret=_INTERP,
    )(t_row, w_row, idx0)


def _rowgather_kernel(perm_hbm, sel_hbm, out_hbm, idx_v, rows_v, sem):
    wid = lax.axis_index("s") * 2 + lax.axis_index("c")
    b = wid * (NSEL // NTILES)
    pltpu.sync_copy(sel_hbm.at[pl.ds(b, NSEL // NTILES)], idx_v)
    pltpu.async_copy(perm_hbm.at[idx_v], rows_v, sem).wait()
    pltpu.sync_copy(rows_v, out_hbm.at[pl.ds(b, NSEL // NTILES)])


def _rowgather_stage(perm, sel):
    mesh = plsc.VectorSubcoreMesh(core_axis_name="c", subcore_axis_name="s",
                                  num_cores=2, num_subcores=16)
    k = pl.kernel(
        _rowgather_kernel,
        out_type=jax.ShapeDtypeStruct((NSEL, NWORDS), jnp.int32),
        mesh=mesh,
        scratch_types=[
            pltpu.VMEM((NSEL // NTILES,), jnp.int32),
            pltpu.VMEM((NSEL // NTILES, NWORDS), jnp.int32),
            pltpu.SemaphoreType.DMA,
        ],
        compiler_params=_SC_PARAMS,
        interpret=_INTERP,
    )
    return k(perm, sel)


def kernel(n_words, bigram, start, end, gumbel_noise, uniforms):
    start2 = start.reshape(1, NWORDS)
    end2 = end.reshape(1, NWORDS)
    perm, pidx, gold, se = _sort_stage(gumbel_noise, start2, end2)
    wpart = _score_stage(bigram.reshape(-1), pidx)
    t_row, w_row, idx0 = _prep_stage(wpart, se.reshape(1, CS),
                                     gold.reshape(1, CS),
                                     uniforms.reshape(1, CS))
    sel = _scan_stage(t_row, w_row, idx0)
    return _rowgather_stage(perm, sel.reshape(NSEL))


# speedup vs baseline: 1.1815x; 1.0043x over previous
import functools

import jax
import jax.numpy as jnp
from jax import lax
from jax.experimental import pallas as pl
from jax.experimental.pallas import tpu as pltpu
from jax.experimental.pallas import tpu_sc as plsc

_INTERP = False
_SC_PARAMS = pltpu.CompilerParams(needs_layout_passes=False)

NWORDS = 512
CS = 10240
BLK = 128
NBLK = CS // BLK
NSEL = CS // 10
NTILES = 32
NQ = 4
NG = NTILES // NQ
CPG = CS // NG
QROWS = NWORDS // NQ
QELEMS = QROWS * NWORDS


def _sort_body(g_ref, start_ref, end_ref, perm_ref, pidx_ref, gold_ref, se_ref):
    keys = g_ref[...]
    lane = lax.broadcasted_iota(jnp.int32, (BLK, NWORDS), 1)
    pay = lane
    for k in range(1, 10):
        kbit = 1 << k
        up = (lane & kbit) == 0
        for j in range(k - 1, -1, -1):
            d = 1 << j
            is_upper = (lane & d) != 0
            pk = jnp.where(is_upper, jnp.roll(keys, d, axis=1),
                           jnp.roll(keys, -d, axis=1))
            pp = jnp.where(is_upper, jnp.roll(pay, d, axis=1),
                           jnp.roll(pay, -d, axis=1))
            own_gt = (keys > pk) | ((keys == pk) & (pay > pp))
            take = jnp.logical_xor(own_gt, jnp.logical_xor(up, is_upper))
            take = jnp.logical_not(take)
            keys = jnp.where(take, pk, keys)
            pay = jnp.where(take, pp, pay)

    perm_ref[...] = pay
    nxt = jnp.roll(pay, -1, axis=1)
    pidx_ref[...] = pay * NWORDS + nxt
    gold_ref[...] = jnp.all(pay == lane, axis=1, keepdims=True).astype(jnp.int32)
    start_b = start_ref[...]
    end_b = end_ref[...]
    se0 = jnp.sum(jnp.where(lane == pay[:, 0:1], start_b, 0.0), axis=1,
                  keepdims=True)
    se1 = jnp.sum(jnp.where(lane == pay[:, NWORDS - 1:NWORDS], end_b, 0.0),
                  axis=1, keepdims=True)
    se_ref[...] = se0 + se1


def _sort_stage(gumbel, start2, end2):
    return pl.pallas_call(
        _sort_body,
        grid=(NBLK,),
        in_specs=[
            pl.BlockSpec((BLK, NWORDS), lambda i: (i, 0)),
            pl.BlockSpec((1, NWORDS), lambda i: (0, 0)),
            pl.BlockSpec((1, NWORDS), lambda i: (0, 0)),
        ],
        out_specs=[
            pl.BlockSpec((BLK, NWORDS), lambda i: (i, 0)),
            pl.BlockSpec((BLK, NWORDS), lambda i: (i, 0)),
            pl.BlockSpec((BLK, 1), lambda i: (i, 0)),
            pl.BlockSpec((BLK, 1), lambda i: (i, 0)),
        ],
        out_shape=[
            jax.ShapeDtypeStruct((CS, NWORDS), jnp.int32),
            jax.ShapeDtypeStruct((CS, NWORDS), jnp.int32),
            jax.ShapeDtypeStruct((CS, 1), jnp.int32),
            jax.ShapeDtypeStruct((CS, 1), jnp.float32),
        ],
        interpret=_INTERP,
    )(gumbel, start2, end2)


SCCHUNK = 32


def _score_kernel(bigf_hbm, pidx_hbm, wpart_hbm, bq, ib0, ib1, wacc,
                  semt, sem0, sem1):
    wid = lax.axis_index("s") * 2 + lax.axis_index("c")
    q = wid % NQ
    g = wid // NQ
    base = q * QELEMS
    ct = pltpu.async_copy(bigf_hbm.at[pl.ds(base, QELEMS)], bq, semt)
    lanes = lax.iota(jnp.int32, 16)
    base_v = jnp.full((16,), base, jnp.int32)
    ct.wait()

    def process(ibuf, wacc_off):
        for cc in range(SCCHUNK // 16):
            lanes_c = lanes + 16 * cc

            def one_j(j, acc):
                jv = jnp.full((16,), 0, jnp.int32) + j
                iv = plsc.load_gather(ibuf, [lanes_c, jv])
                li = iv - base_v
                ok = (li >= 0) & (li < QELEMS)
                lic = li & (QELEMS - 1)
                v = plsc.load_gather(bq, [lic], mask=ok)
                return acc + jnp.where(ok, v, 0.0)

            def jblock(jb, accs):
                accs = list(accs)
                for jj in range(16):
                    accs[jj % 8] = one_j(jb * 16 + jj, accs[jj % 8])
                return tuple(accs)

            accs = (jnp.zeros((16,), jnp.float32),) * 8
            accs = lax.fori_loop(0, 31, jblock, accs)
            accs = list(accs)
            for jj in range(496, 511):
                accs[jj % 8] = one_j(jj, accs[jj % 8])
            acc = (((accs[0] + accs[1]) + (accs[2] + accs[3]))
                   + ((accs[4] + accs[5]) + (accs[6] + accs[7])))
            wacc[pl.ds(wacc_off + cc * 16, 16)] = acc

    @pl.loop(0, CPG // (2 * SCCHUNK))
    def _pair(h):
        cbase = g * CPG + h * 2 * SCCHUNK
        cpa = pltpu.async_copy(pidx_hbm.at[pl.ds(cbase, SCCHUNK)],
                               ib0.at[:, pl.ds(0, NWORDS)], sem0)
        cpb = pltpu.async_copy(pidx_hbm.at[pl.ds(cbase + SCCHUNK, SCCHUNK)],
                               ib1.at[:, pl.ds(0, NWORDS)], sem1)
        cpa.wait()
        process(ib0, h * 2 * SCCHUNK)
        cpb.wait()
        process(ib1, h * 2 * SCCHUNK + SCCHUNK)

    pltpu.sync_copy(wacc, wpart_hbm.at[q, pl.ds(g * CPG, CPG)])


def _score_stage(bigf, pidx):
    mesh = plsc.VectorSubcoreMesh(core_axis_name="c", subcore_axis_name="s",
                                  num_cores=2, num_subcores=16)
    k = pl.kernel(
        _score_kernel,
        out_type=jax.ShapeDtypeStruct((NQ, CS), jnp.float32),
        mesh=mesh,
        scratch_types=[
            pltpu.VMEM((QELEMS,), jnp.float32),
            pltpu.VMEM((SCCHUNK, NWORDS + 1), jnp.int32),
            pltpu.VMEM((SCCHUNK, NWORDS + 1), jnp.int32),
            pltpu.VMEM((CPG,), jnp.float32),
            pltpu.SemaphoreType.DMA,
            pltpu.SemaphoreType.DMA,
            pltpu.SemaphoreType.DMA,
        ],
        compiler_params=_SC_PARAMS,
        interpret=_INTERP,
    )
    return k(bigf, pidx)


def _prep_body(wpart_ref, se_ref, gold_ref, u_ref, t_ref, w_ref, idx0_ref):
    wp = wpart_ref[...]
    w = wp[0:1] + wp[1:2] + wp[2:3] + wp[3:4] + se_ref[...]
    lu = jnp.log(u_ref[...])
    lane = lax.broadcasted_iota(jnp.int32, (1, CS), 1)
    gold = gold_ref[...] != 0
    neg_inf = jnp.float32(-jnp.inf)
    t = jnp.where(gold | (lane == 0), neg_inf, w - lu)
    t_ref[...] = t
    w_ref[...] = w
    idx0 = jnp.min(jnp.where(gold, CS, lane))
    idx0_ref[0, 0] = jnp.where(idx0 == CS, 0, idx0).astype(jnp.int32)


def _prep_stage(wpart, se_row, gold_row, u_row):
    return pl.pallas_call(
        _prep_body,
        out_shape=[
            jax.ShapeDtypeStruct((1, CS), jnp.float32),
            jax.ShapeDtypeStruct((1, CS), jnp.float32),
            jax.ShapeDtypeStruct((1, 1), jnp.int32),
        ],
        in_specs=[
            pl.BlockSpec((NQ, CS), lambda: (0, 0)),
            pl.BlockSpec((1, CS), lambda: (0, 0)),
            pl.BlockSpec((1, CS), lambda: (0, 0)),
            pl.BlockSpec((1, CS), lambda: (0, 0)),
        ],
        out_specs=[
            pl.BlockSpec((1, CS), lambda: (0, 0)),
            pl.BlockSpec((1, CS), lambda: (0, 0)),
            pl.BlockSpec(memory_space=pltpu.MemorySpace.SMEM),
        ],
        interpret=_INTERP,
    )(wpart, se_row, gold_row, u_row)


def _scan_body(t_ref, w_ref, idx0_ref, sel_ref):
    a0 = idx0_ref[0, 0]
    w0 = w_ref[0, 0]

    def outer(s, carry):
        a, wl = carry
        for r in range(10):
            i = s * 10 + r
            ti = t_ref[0, i]
            acc = ti > wl
            a = jnp.where(acc, i, a)
            wl = jnp.where(acc, w_ref[0, i], wl)
            if r == 9:
                sel_ref[0, s] = a
        return a, wl

    lax.fori_loop(0, NSEL, outer, (a0, w0))


def _scan_stage(t_row, w_row, idx0):
    return pl.pallas_call(
        _scan_body,
        out_shape=jax.ShapeDtypeStruct((1, NSEL), jnp.int32),
        in_specs=[
            pl.BlockSpec(memory_space=pltpu.MemorySpace.SMEM),
            pl.BlockSpec(memory_space=pltpu.MemorySpace.SMEM),
            pl.BlockSpec(memory_space=pltpu.MemorySpace.SMEM),
        ],
        out_specs=pl.BlockSpec(memory_space=pltpu.MemorySpace.SMEM),
        interpret=_INTERP,
    )(t_row, w_row, idx0)


def _rowgather_kernel(perm_hbm, sel_hbm, out_hbm, idx_v, rows_v, sem):
    wid = lax.axis_index("s") * 2 + lax.axis_index("c")
    b = wid * (NSEL // NTILES)
    pltpu.sync_copy(sel_hbm.at[pl.ds(b, NSEL // NTILES)], idx_v)
    pltpu.async_copy(perm_hbm.at[idx_v], rows_v, sem).wait()
    pltpu.sync_copy(rows_v, out_hbm.at[pl.ds(b, NSEL // NTILES)])


def _rowgather_stage(perm, sel):
    mesh = plsc.VectorSubcoreMesh(core_axis_name="c", subcore_axis_name="s",
                                  num_cores=2, num_subcores=16)
    k = pl.kernel(
        _rowgather_kernel,
        out_type=jax.ShapeDtypeStruct((NSEL, NWORDS), jnp.int32),
        mesh=mesh,
        scratch_types=[
            pltpu.VMEM((NSEL // NTILES,), jnp.int32),
            pltpu.VMEM((NSEL // NTILES, NWORDS), jnp.int32),
            pltpu.SemaphoreType.DMA,
        ],
        compiler_params=_SC_PARAMS,
        interpret=_INTERP,
    )
    return k(perm, sel)


def kernel(n_words, bigram, start, end, gumbel_noise, uniforms):
    start2 = start.reshape(1, NWORDS)
    end2 = end.reshape(1, NWORDS)
    perm, pidx, gold, se = _sort_stage(gumbel_noise, start2, end2)
    wpart = _score_stage(bigram.reshape(-1), pidx)
    t_row, w_row, idx0 = _prep_stage(wpart, se.reshape(1, CS),
                                     gold.reshape(1, CS),
                                     uniforms.reshape(1, CS))
    sel = _scan_stage(t_row, w_row, idx0)
    return _rowgather_stage(perm, sel.reshape(NSEL))
